# baseline (speedup 1.0000x reference)
import functools

import jax
import jax.numpy as jnp
from jax import lax
from jax.experimental import pallas as pl
from jax.experimental.pallas import tpu as pltpu
from jax.experimental.pallas import tpu_sc as plsc


def _sc_gather(flat_idx, table, n_rows, emb):
    info = plsc.get_sparse_core_info()
    nc, ns = info.num_cores, info.num_subcores
    nw = nc * ns
    per_w = n_rows // nw
    n_chunks = 2
    ch = per_w // n_chunks

    mesh = plsc.VectorSubcoreMesh(core_axis_name="c", subcore_axis_name="s")

    @functools.partial(
        pl.kernel,
        mesh=mesh,
        out_type=jax.ShapeDtypeStruct((n_rows, emb), jnp.float32),
        scratch_types=[
            pltpu.VMEM((per_w,), jnp.int32),
            pltpu.VMEM((ch, emb), jnp.float32),
            pltpu.SemaphoreType.DMA,
        ],
        compiler_params=pltpu.CompilerParams(use_tc_tiling_on_sc=False),
    )
    def gather_kernel(idx_hbm, tab_hbm, out_hbm, idx_v, rows_v, sem):
        wid = lax.axis_index("s") * nc + lax.axis_index("c")
        base = wid * per_w
        pltpu.sync_copy(idx_hbm.at[pl.ds(base, per_w)], idx_v)
        for c in range(n_chunks):
            pltpu.async_copy(
                tab_hbm.at[idx_v.at[pl.ds(c * ch, ch)]], rows_v, sem
            ).wait()
            pltpu.sync_copy(rows_v, out_hbm.at[pl.ds(base + c * ch, ch)])

    return gather_kernel(flat_idx, table)


def _mlp_body(g_ref, w1_ref, b1_ref, w2_ref, b2_ref, out_ref):
    g = g_ref[...].astype(jnp.bfloat16)
    big = jnp.concatenate([g, jnp.roll(g, -1, axis=0)], axis=1)
    p = jnp.dot(big, w1_ref[...], preferred_element_type=jnp.float32)
    h = jnp.tanh((p + b1_ref[...]).astype(jnp.bfloat16))
    o = jnp.dot(h, w2_ref[...], preferred_element_type=jnp.float32)
    out_ref[...] = jax.nn.sigmoid(o + b2_ref[...]).T


def kernel(inputs, embed, W1, b1, W2, b2):
    B, L = inputs.shape
    V, E = embed.shape
    SE, HID = W1.shape
    S = SE // E
    adj = L - S + 1
    n_rows = B * L
    per_row = 128 // E

    t4 = embed.reshape(V * E // 128, 128)
    t4 = t4.at[0, :E].set(0.0)
    flat_idx = inputs.reshape(n_rows)
    G = _sc_gather(flat_idx, t4.reshape(V, E), n_rows, E)
    G4 = G.reshape(n_rows // per_row, 128)

    Wbig = jnp.zeros((2 * 128, per_row * HID), jnp.float32)
    W2sel = jnp.zeros((per_row * HID, per_row), jnp.float32)
    for k in range(per_row):
        Wbig = Wbig.at[E * k : E * k + SE, HID * k : HID * k + HID].set(W1)
        W2sel = W2sel.at[HID * k : HID * k + HID, k].set(W2[:, 0])
    Wbig = Wbig.astype(jnp.bfloat16)
    W2sel = W2sel.astype(jnp.bfloat16)
    b1t = jnp.tile(b1, per_row).reshape(1, per_row * HID)
    b2r = b2.reshape(1, 1)

    seq_per_blk = 512
    R4 = seq_per_blk * L // per_row
    nb = n_rows // (R4 * per_row)

    out = pl.pallas_call(
        _mlp_body,
        grid=(nb,),
        in_specs=[
            pl.BlockSpec((R4, 128), lambda i: (i, 0)),
            pl.BlockSpec((2 * 128, per_row * HID), lambda i: (0, 0)),
            pl.BlockSpec((1, per_row * HID), lambda i: (0, 0)),
            pl.BlockSpec((per_row * HID, per_row), lambda i: (0, 0)),
            pl.BlockSpec((1, 1), lambda i: (0, 0)),
        ],
        out_specs=pl.BlockSpec((per_row, R4), lambda i: (0, i)),
        out_shape=jax.ShapeDtypeStruct((per_row, n_rows // per_row), jnp.float32),
        compiler_params=pltpu.CompilerParams(
            dimension_semantics=("arbitrary",),
        ),
    )(G4, Wbig, b1t, W2sel, b2r)

    return out.T.reshape(B, L)[:, :adj]


# speedup vs baseline: 1.0234x; 1.0234x over previous
import functools

import jax
import jax.numpy as jnp
from jax import lax
from jax.experimental import pallas as pl
from jax.experimental.pallas import tpu as pltpu
from jax.experimental.pallas import tpu_sc as plsc


def _sc_gather(flat_idx, table, n_rows, emb):
    info = plsc.get_sparse_core_info()
    nc, ns = info.num_cores, info.num_subcores
    nw = nc * ns
    per_w = n_rows // nw
    n_chunks = 2
    ch = per_w // n_chunks

    mesh = plsc.VectorSubcoreMesh(core_axis_name="c", subcore_axis_name="s")

    @functools.partial(
        pl.kernel,
        mesh=mesh,
        out_type=jax.ShapeDtypeStruct((n_rows, emb), jnp.float32),
        scratch_types=[
            pltpu.VMEM((per_w,), jnp.int32),
            pltpu.VMEM((ch, emb), jnp.float32),
            pltpu.SemaphoreType.DMA,
        ],
        compiler_params=pltpu.CompilerParams(use_tc_tiling_on_sc=False),
    )
    def gather_kernel(idx_hbm, tab_hbm, out_hbm, idx_v, rows_v, sem):
        wid = lax.axis_index("s") * nc + lax.axis_index("c")
        base = wid * per_w
        pltpu.sync_copy(idx_hbm.at[pl.ds(base, per_w)], idx_v)
        for c in range(n_chunks):
            pltpu.async_copy(
                tab_hbm.at[idx_v.at[pl.ds(c * ch, ch)]], rows_v, sem
            ).wait()
            pltpu.sync_copy(rows_v, out_hbm.at[pl.ds(base + c * ch, ch)])

    return gather_kernel(flat_idx, table)


def _mlp_body(g_ref, w1_ref, b1_ref, w2_ref, b2_ref, out_ref):
    g = g_ref[...].astype(jnp.bfloat16)
    big = jnp.concatenate([g, jnp.roll(g, -1, axis=0)], axis=1)
    p = jnp.dot(big, w1_ref[...], preferred_element_type=jnp.float32)
    h = jnp.tanh((p + b1_ref[...]).astype(jnp.bfloat16))
    o = jnp.dot(h, w2_ref[...], preferred_element_type=jnp.float32)
    out_ref[...] = jax.nn.sigmoid(o + b2_ref[...])


def kernel(inputs, embed, W1, b1, W2, b2):
    B, L = inputs.shape
    V, E = embed.shape
    SE, HID = W1.shape
    S = SE // E
    adj = L - S + 1
    n_rows = B * L
    per_row = 128 // E

    t4 = embed.reshape(V * E // 128, 128)
    t4 = t4.at[0, :E].set(0.0)
    flat_idx = inputs.reshape(n_rows)
    G = _sc_gather(flat_idx, t4.reshape(V, E), n_rows, E)
    G4 = G.reshape(n_rows // per_row, 128)

    Wbig = jnp.zeros((2 * 128, per_row * HID), jnp.float32)
    W2sel = jnp.zeros((per_row * HID, per_row), jnp.float32)
    for k in range(per_row):
        Wbig = Wbig.at[E * k : E * k + SE, HID * k : HID * k + HID].set(W1)
        W2sel = W2sel.at[HID * k : HID * k + HID, k].set(W2[:, 0])
    Wbig = Wbig.astype(jnp.bfloat16)
    W2sel = W2sel.astype(jnp.bfloat16)
    b1t = jnp.tile(b1, per_row).reshape(1, per_row * HID)
    b2r = b2.reshape(1, 1)

    seq_per_blk = 512
    R4 = seq_per_blk * L // per_row
    nb = n_rows // (R4 * per_row)

    out = pl.pallas_call(
        _mlp_body,
        grid=(nb,),
        in_specs=[
            pl.BlockSpec((R4, 128), lambda i: (i, 0)),
            pl.BlockSpec((2 * 128, per_row * HID), lambda i: (0, 0)),
            pl.BlockSpec((1, per_row * HID), lambda i: (0, 0)),
            pl.BlockSpec((per_row * HID, per_row), lambda i: (0, 0)),
            pl.BlockSpec((1, 1), lambda i: (0, 0)),
        ],
        out_specs=pl.BlockSpec((R4, per_row), lambda i: (i, 0)),
        out_shape=jax.ShapeDtypeStruct((n_rows // per_row, per_row), jnp.float32),
        compiler_params=pltpu.CompilerParams(
            dimension_semantics=("arbitrary",),
        ),
    )(G4, Wbig, b1t, W2sel, b2r)

    return out.reshape(B, L)[:, :adj]
